# Initial kernel scaffold; baseline (speedup 1.0000x reference)
import jax
import jax.numpy as jnp
from jax.experimental import pallas as pl


def kernel(h, adj, W1, a_src1, a_dst1, W2, a_src2, a_dst2, Ws1, Ws2, Wd, bd):
    raise NotImplementedError("write your pallas kernel here")


# speedup vs baseline: 1.4231x; 1.4231x over previous
import jax
import jax.numpy as jnp
from jax.experimental import pallas as pl

_B, _N, _F0 = 4, 1024, 11
_H, _F1, _F2 = 6, 32, 64
_D1 = _H * _F1
_D2 = _H * _F2


def _dotT(a, b):
    return jax.lax.dot_general(a, b, (((1,), (1,)), ((), ())),
                               preferred_element_type=jnp.float32)


def _dotN(a, b):
    return jax.lax.dot_general(a, b, (((0,), (0,)), ((), ())),
                               preferred_element_type=jnp.float32)


def _gat_head(mask, Wh, src_col, dst_row):
    e = src_col + dst_row
    e = jnp.where(e >= 0, e, 0.2 * e)
    e = jnp.where(mask, e, jnp.float32(-1e9))
    m = jnp.max(e, axis=1, keepdims=True)
    p = jnp.exp(e - m)
    s = jnp.sum(p, axis=1, keepdims=True)
    o = jnp.dot(p, Wh, preferred_element_type=jnp.float32) / s
    return jnp.where(o > 0, o, jnp.exp(jnp.minimum(o, 0.0)) - 1.0)


def _layer1(h_ref, adj_ref, W_ref, as_ref, ad_ref, out_ref):
    hb = h_ref[0]
    mask = adj_ref[0] > 0.5
    Wh = jnp.dot(hb, W_ref[0], preferred_element_type=jnp.float32)
    src_col = jnp.dot(Wh, as_ref[0], preferred_element_type=jnp.float32)
    dst_row = _dotT(ad_ref[0], Wh)
    out_ref[0, 0] = _gat_head(mask, Wh, src_col, dst_row)


def _layer2(x_ref, adj_ref, W_ref, as_ref, ad_ref, out_ref):
    mask = adj_ref[0] > 0.5
    Wh = jnp.dot(x_ref[0, 0], W_ref[0, 0:_F1, :],
                 preferred_element_type=jnp.float32)
    for j in range(1, _H):
        Wh += jnp.dot(x_ref[0, j], W_ref[0, j * _F1:(j + 1) * _F1, :],
                      preferred_element_type=jnp.float32)
    src_col = jnp.dot(Wh, as_ref[0], preferred_element_type=jnp.float32)
    dst_row = _dotT(ad_ref[0], Wh)
    out_ref[0, 0] = _gat_head(mask, Wh, src_col, dst_row)


def _pool(x_ref, Ws1_ref, Ws2_ref, WdR_ref, out_ref):
    u = jnp.dot(x_ref[0, 0], Ws1_ref[0:_F2, :],
                preferred_element_type=jnp.float32)
    for j in range(1, _H):
        u += jnp.dot(x_ref[0, j], Ws1_ref[j * _F2:(j + 1) * _F2, :],
                     preferred_element_type=jnp.float32)
    u = jnp.tanh(u)
    scores = _dotT(u, Ws2_ref[...])
    m = jnp.max(scores, axis=0, keepdims=True)
    p = jnp.exp(scores - m)
    A = p / jnp.sum(p, axis=0, keepdims=True)
    val = jnp.float32(0.0)
    for j in range(_H):
        Mj = _dotN(A, x_ref[0, j])
        val += jnp.sum(Mj * WdR_ref[:, j * _F2:(j + 1) * _F2])
    out_ref[...] = jnp.zeros((1, 8, 128), jnp.float32) + val


def kernel(h, adj, W1, a_src1, a_dst1, W2, a_src2, a_dst2, Ws1, Ws2, Wd, bd):
    B, N, F0 = h.shape
    as1 = a_src1[:, :, None]
    ad1 = a_dst1[:, None, :]
    as2 = a_src2[:, :, None]
    ad2 = a_dst2[:, None, :]
    WdR = Wd.reshape(_H, _D2)

    x1 = pl.pallas_call(
        _layer1,
        grid=(B, _H),
        in_specs=[
            pl.BlockSpec((1, N, F0), lambda b, k: (b, 0, 0)),
            pl.BlockSpec((1, N, N), lambda b, k: (b, 0, 0)),
            pl.BlockSpec((1, F0, _F1), lambda b, k: (k, 0, 0)),
            pl.BlockSpec((1, _F1, 1), lambda b, k: (k, 0, 0)),
            pl.BlockSpec((1, 1, _F1), lambda b, k: (k, 0, 0)),
        ],
        out_specs=pl.BlockSpec((1, 1, N, _F1), lambda b, k: (b, k, 0, 0)),
        out_shape=jax.ShapeDtypeStruct((B, _H, N, _F1), jnp.float32),
    )(h, adj, W1, as1, ad1)

    x2 = pl.pallas_call(
        _layer2,
        grid=(B, _H),
        in_specs=[
            pl.BlockSpec((1, _H, N, _F1), lambda b, k: (b, 0, 0, 0)),
            pl.BlockSpec((1, N, N), lambda b, k: (b, 0, 0)),
            pl.BlockSpec((1, _D1, _F2), lambda b, k: (k, 0, 0)),
            pl.BlockSpec((1, _F2, 1), lambda b, k: (k, 0, 0)),
            pl.BlockSpec((1, 1, _F2), lambda b, k: (k, 0, 0)),
        ],
        out_specs=pl.BlockSpec((1, 1, N, _F2), lambda b, k: (b, k, 0, 0)),
        out_shape=jax.ShapeDtypeStruct((B, _H, N, _F2), jnp.float32),
    )(x1, adj, W2, as2, ad2)

    out = pl.pallas_call(
        _pool,
        grid=(B,),
        in_specs=[
            pl.BlockSpec((1, _H, N, _F2), lambda b: (b, 0, 0, 0)),
            pl.BlockSpec((_D2, _D2), lambda b: (0, 0)),
            pl.BlockSpec((_H, _D2), lambda b: (0, 0)),
            pl.BlockSpec((_H, _D2), lambda b: (0, 0)),
        ],
        out_specs=pl.BlockSpec((1, 8, 128), lambda b: (b, 0, 0)),
        out_shape=jax.ShapeDtypeStruct((B, 8, 128), jnp.float32),
    )(x2, Ws1, Ws2, WdR)
    return out[:, 0, 0] + bd[0]


# speedup vs baseline: 1.4967x; 1.0517x over previous
import jax
import jax.numpy as jnp
from jax.experimental import pallas as pl
from jax.experimental.pallas import tpu as pltpu

_B, _N, _F0 = 4, 1024, 11
_H, _F1, _F2 = 6, 32, 64
_D1 = _H * _F1
_D2 = _H * _F2


def _dotT(a, b):
    return jax.lax.dot_general(a, b, (((1,), (1,)), ((), ())),
                               preferred_element_type=jnp.float32)


def _dotN(a, b):
    return jax.lax.dot_general(a, b, (((0,), (0,)), ((), ())),
                               preferred_element_type=jnp.float32)


def _mask_bias(adj_ref, bias_ref):
    @pl.when(pl.program_id(1) == 0)
    def _():
        bias_ref[...] = jnp.where(adj_ref[0] > 0.5, jnp.float32(0.0),
                                  jnp.float32(-1e9))


def _gat_head(bias, Wh, src_col, dst_row):
    e = src_col + dst_row
    e = jnp.maximum(e, 0.2 * e) + bias
    m = jnp.max(e, axis=1, keepdims=True)
    p = jnp.exp(e - m)
    ones = jnp.ones((_N, 1), jnp.float32)
    num = jnp.dot(p, jnp.concatenate([Wh, ones], axis=1),
                  preferred_element_type=jnp.float32)
    o = num[:, :-1] / num[:, -1:]
    return jnp.where(o > 0, o, jnp.exp(jnp.minimum(o, 0.0)) - 1.0)


def _layer1(h_ref, adj_ref, W_ref, as_ref, ad_ref, out_ref, bias_ref):
    _mask_bias(adj_ref, bias_ref)
    Wh = jnp.dot(h_ref[0], W_ref[0], preferred_element_type=jnp.float32)
    src_col = jnp.dot(Wh, as_ref[0], preferred_element_type=jnp.float32)
    dst_row = _dotT(ad_ref[0], Wh)
    out_ref[0, 0] = _gat_head(bias_ref[...], Wh, src_col, dst_row)


def _layer2(x_ref, adj_ref, W_ref, as_ref, ad_ref, out_ref, bias_ref):
    _mask_bias(adj_ref, bias_ref)
    Wh = jnp.dot(x_ref[0, 0], W_ref[0, 0:_F1, :],
                 preferred_element_type=jnp.float32)
    for j in range(1, _H):
        Wh += jnp.dot(x_ref[0, j], W_ref[0, j * _F1:(j + 1) * _F1, :],
                      preferred_element_type=jnp.float32)
    src_col = jnp.dot(Wh, as_ref[0], preferred_element_type=jnp.float32)
    dst_row = _dotT(ad_ref[0], Wh)
    out_ref[0, 0] = _gat_head(bias_ref[...], Wh, src_col, dst_row)


def _pool(x_ref, Ws1_ref, Ws2_ref, WdR_ref, out_ref):
    u = jnp.dot(x_ref[0, 0], Ws1_ref[0:_F2, :],
                preferred_element_type=jnp.float32)
    for j in range(1, _H):
        u += jnp.dot(x_ref[0, j], Ws1_ref[j * _F2:(j + 1) * _F2, :],
                     preferred_element_type=jnp.float32)
    u = jnp.tanh(u)
    scores = _dotT(u, Ws2_ref[...])
    m = jnp.max(scores, axis=0, keepdims=True)
    p = jnp.exp(scores - m)
    A = p / jnp.sum(p, axis=0, keepdims=True)
    val = jnp.float32(0.0)
    for j in range(_H):
        Mj = _dotN(A, x_ref[0, j])
        val += jnp.sum(Mj * WdR_ref[:, j * _F2:(j + 1) * _F2])
    out_ref[...] = jnp.zeros((1, 8, 128), jnp.float32) + val


def kernel(h, adj, W1, a_src1, a_dst1, W2, a_src2, a_dst2, Ws1, Ws2, Wd, bd):
    B, N, F0 = h.shape
    as1 = a_src1[:, :, None]
    ad1 = a_dst1[:, None, :]
    as2 = a_src2[:, :, None]
    ad2 = a_dst2[:, None, :]
    WdR = Wd.reshape(_H, _D2)

    scratch = [pltpu.VMEM((N, N), jnp.float32)]

    x1 = pl.pallas_call(
        _layer1,
        grid=(B, _H),
        in_specs=[
            pl.BlockSpec((1, N, F0), lambda b, k: (b, 0, 0)),
            pl.BlockSpec((1, N, N), lambda b, k: (b, 0, 0)),
            pl.BlockSpec((1, F0, _F1), lambda b, k: (k, 0, 0)),
            pl.BlockSpec((1, _F1, 1), lambda b, k: (k, 0, 0)),
            pl.BlockSpec((1, 1, _F1), lambda b, k: (k, 0, 0)),
        ],
        out_specs=pl.BlockSpec((1, 1, N, _F1), lambda b, k: (b, k, 0, 0)),
        out_shape=jax.ShapeDtypeStruct((B, _H, N, _F1), jnp.float32),
        scratch_shapes=scratch,
    )(h, adj, W1, as1, ad1)

    x2 = pl.pallas_call(
        _layer2,
        grid=(B, _H),
        in_specs=[
            pl.BlockSpec((1, _H, N, _F1), lambda b, k: (b, 0, 0, 0)),
            pl.BlockSpec((1, N, N), lambda b, k: (b, 0, 0)),
            pl.BlockSpec((1, _D1, _F2), lambda b, k: (k, 0, 0)),
            pl.BlockSpec((1, _F2, 1), lambda b, k: (k, 0, 0)),
            pl.BlockSpec((1, 1, _F2), lambda b, k: (k, 0, 0)),
        ],
        out_specs=pl.BlockSpec((1, 1, N, _F2), lambda b, k: (b, k, 0, 0)),
        out_shape=jax.ShapeDtypeStruct((B, _H, N, _F2), jnp.float32),
        scratch_shapes=scratch,
    )(x1, adj, W2, as2, ad2)

    out = pl.pallas_call(
        _pool,
        grid=(B,),
        in_specs=[
            pl.BlockSpec((1, _H, N, _F2), lambda b: (b, 0, 0, 0)),
            pl.BlockSpec((_D2, _D2), lambda b: (0, 0)),
            pl.BlockSpec((_H, _D2), lambda b: (0, 0)),
            pl.BlockSpec((_H, _D2), lambda b: (0, 0)),
        ],
        out_specs=pl.BlockSpec((1, 8, 128), lambda b: (b, 0, 0)),
        out_shape=jax.ShapeDtypeStruct((B, 8, 128), jnp.float32),
    )(x2, Ws1, Ws2, WdR)
    return out[:, 0, 0] + bd[0]


# speedup vs baseline: 1.6524x; 1.1040x over previous
import jax
import jax.numpy as jnp
from jax.experimental import pallas as pl
from jax.experimental.pallas import tpu as pltpu

_B, _N, _F0 = 4, 1024, 11
_H, _F1, _F2 = 6, 32, 64
_D1 = _H * _F1
_D2 = _H * _F2


def _dotT(a, b):
    return jax.lax.dot_general(a, b, (((1,), (1,)), ((), ())),
                               preferred_element_type=jnp.float32)


def _dotN(a, b):
    return jax.lax.dot_general(a, b, (((0,), (0,)), ((), ())),
                               preferred_element_type=jnp.float32)


def _mask_bias(adj_ref, bias_ref):
    @pl.when(pl.program_id(1) == 0)
    def _():
        bias_ref[...] = jnp.where(adj_ref[0] > 0.5, jnp.float32(0.0),
                                  jnp.float32(-1e9))


def _gat_head(bias, Wh, src_col, dst_row):
    e = src_col + dst_row
    e = jnp.maximum(e, 0.2 * e) + bias
    p = jnp.exp(jnp.minimum(e, 60.0)).astype(jnp.bfloat16)
    ones = jnp.ones((_N, 1), jnp.bfloat16)
    num = jnp.dot(p, jnp.concatenate([Wh.astype(jnp.bfloat16), ones], axis=1),
                  preferred_element_type=jnp.float32)
    o = num[:, :-1] / num[:, -1:]
    return jnp.where(o > 0, o, jnp.exp(jnp.minimum(o, 0.0)) - 1.0)


def _layer1(h_ref, adj_ref, W_ref, as_ref, ad_ref, out_ref, bias_ref):
    _mask_bias(adj_ref, bias_ref)
    Wh = jnp.dot(h_ref[0], W_ref[0], preferred_element_type=jnp.float32)
    src_col = jnp.dot(Wh, as_ref[0], preferred_element_type=jnp.float32)
    dst_row = _dotT(ad_ref[0], Wh)
    out_ref[0, 0] = _gat_head(bias_ref[...], Wh, src_col, dst_row)


def _layer2(x_ref, adj_ref, W_ref, as_ref, ad_ref, out_ref, bias_ref):
    _mask_bias(adj_ref, bias_ref)
    Wh = jnp.dot(x_ref[0, 0], W_ref[0, 0:_F1, :],
                 preferred_element_type=jnp.float32)
    for j in range(1, _H):
        Wh += jnp.dot(x_ref[0, j], W_ref[0, j * _F1:(j + 1) * _F1, :],
                      preferred_element_type=jnp.float32)
    src_col = jnp.dot(Wh, as_ref[0], preferred_element_type=jnp.float32)
    dst_row = _dotT(ad_ref[0], Wh)
    out_ref[0, 0] = _gat_head(bias_ref[...], Wh, src_col, dst_row)


def _pool(x_ref, Ws1_ref, Ws2_ref, WdR_ref, out_ref):
    u = jnp.dot(x_ref[0, 0], Ws1_ref[0:_F2, :],
                preferred_element_type=jnp.float32)
    for j in range(1, _H):
        u += jnp.dot(x_ref[0, j], Ws1_ref[j * _F2:(j + 1) * _F2, :],
                     preferred_element_type=jnp.float32)
    u = jnp.tanh(u)
    scores = _dotT(u, Ws2_ref[...])
    m = jnp.max(scores, axis=0, keepdims=True)
    p = jnp.exp(scores - m)
    A = p / jnp.sum(p, axis=0, keepdims=True)
    val = jnp.float32(0.0)
    for j in range(_H):
        Mj = _dotN(A, x_ref[0, j])
        val += jnp.sum(Mj * WdR_ref[:, j * _F2:(j + 1) * _F2])
    out_ref[...] = jnp.zeros((1, 8, 128), jnp.float32) + val


def kernel(h, adj, W1, a_src1, a_dst1, W2, a_src2, a_dst2, Ws1, Ws2, Wd, bd):
    B, N, F0 = h.shape
    as1 = a_src1[:, :, None]
    ad1 = a_dst1[:, None, :]
    as2 = a_src2[:, :, None]
    ad2 = a_dst2[:, None, :]
    WdR = Wd.reshape(_H, _D2)

    scratch = [pltpu.VMEM((N, N), jnp.float32)]

    x1 = pl.pallas_call(
        _layer1,
        grid=(B, _H),
        in_specs=[
            pl.BlockSpec((1, N, F0), lambda b, k: (b, 0, 0)),
            pl.BlockSpec((1, N, N), lambda b, k: (b, 0, 0)),
            pl.BlockSpec((1, F0, _F1), lambda b, k: (k, 0, 0)),
            pl.BlockSpec((1, _F1, 1), lambda b, k: (k, 0, 0)),
            pl.BlockSpec((1, 1, _F1), lambda b, k: (k, 0, 0)),
        ],
        out_specs=pl.BlockSpec((1, 1, N, _F1), lambda b, k: (b, k, 0, 0)),
        out_shape=jax.ShapeDtypeStruct((B, _H, N, _F1), jnp.float32),
        scratch_shapes=scratch,
    )(h, adj, W1, as1, ad1)

    x2 = pl.pallas_call(
        _layer2,
        grid=(B, _H),
        in_specs=[
            pl.BlockSpec((1, _H, N, _F1), lambda b, k: (b, 0, 0, 0)),
            pl.BlockSpec((1, N, N), lambda b, k: (b, 0, 0)),
            pl.BlockSpec((1, _D1, _F2), lambda b, k: (k, 0, 0)),
            pl.BlockSpec((1, _F2, 1), lambda b, k: (k, 0, 0)),
            pl.BlockSpec((1, 1, _F2), lambda b, k: (k, 0, 0)),
        ],
        out_specs=pl.BlockSpec((1, 1, N, _F2), lambda b, k: (b, k, 0, 0)),
        out_shape=jax.ShapeDtypeStruct((B, _H, N, _F2), jnp.float32),
        scratch_shapes=scratch,
    )(x1, adj, W2, as2, ad2)

    out = pl.pallas_call(
        _pool,
        grid=(B,),
        in_specs=[
            pl.BlockSpec((1, _H, N, _F2), lambda b: (b, 0, 0, 0)),
            pl.BlockSpec((_D2, _D2), lambda b: (0, 0)),
            pl.BlockSpec((_H, _D2), lambda b: (0, 0)),
            pl.BlockSpec((_H, _D2), lambda b: (0, 0)),
        ],
        out_specs=pl.BlockSpec((1, 8, 128), lambda b: (b, 0, 0)),
        out_shape=jax.ShapeDtypeStruct((B, 8, 128), jnp.float32),
    )(x2, Ws1, Ws2, WdR)
    return out[:, 0, 0] + bd[0]


# speedup vs baseline: 1.7682x; 1.0701x over previous
import jax
import jax.numpy as jnp
from jax.experimental import pallas as pl
from jax.experimental.pallas import tpu as pltpu

_B, _N, _F0 = 4, 1024, 11
_H, _F1, _F2 = 6, 32, 64
_D1 = _H * _F1
_D2 = _H * _F2


def _dotT(a, b):
    return jax.lax.dot_general(a, b, (((1,), (1,)), ((), ())),
                               preferred_element_type=jnp.float32)


def _dotN(a, b):
    return jax.lax.dot_general(a, b, (((0,), (0,)), ((), ())),
                               preferred_element_type=jnp.float32)


def _gat_head(bias, Wh, src_col, dst_row):
    e = src_col + dst_row
    e = jnp.minimum(jnp.maximum(e, 0.2 * e) + bias, 60.0)
    p = jnp.exp(e).astype(jnp.bfloat16)
    ones = jnp.ones((_N, 1), jnp.bfloat16)
    num = jnp.dot(p, jnp.concatenate([Wh.astype(jnp.bfloat16), ones], axis=1),
                  preferred_element_type=jnp.float32)
    o = num[:, :-1] / num[:, -1:]
    return jnp.where(o > 0, o, jnp.exp(jnp.minimum(o, 0.0)) - 1.0)


def _fused(h_ref, adj_ref, W1_ref, as1_ref, ad1_ref, W2_ref, as2_ref,
           ad2_ref, Ws1_ref, Ws2_ref, WdR_ref, out_ref,
           bias_ref, x1_ref, x2_ref):
    phase = pl.program_id(1)
    k = pl.program_id(2)

    @pl.when(jnp.logical_and(phase == 0, k == 0))
    def _():
        bias_ref[...] = jnp.where(adj_ref[0] > 0.5, jnp.float32(0.0),
                                  jnp.float32(-1e9))

    @pl.when(phase == 0)
    def _():
        Wh = jnp.dot(h_ref[0], W1_ref[k], preferred_element_type=jnp.float32)
        src_col = jnp.dot(Wh, as1_ref[k], preferred_element_type=jnp.float32)
        dst_row = _dotT(ad1_ref[k], Wh)
        x1_ref[k] = _gat_head(bias_ref[...], Wh, src_col, dst_row)

    @pl.when(phase == 1)
    def _():
        Wh = jnp.dot(x1_ref[0], W2_ref[k, 0:_F1, :],
                     preferred_element_type=jnp.float32)
        for j in range(1, _H):
            Wh += jnp.dot(x1_ref[j], W2_ref[k, j * _F1:(j + 1) * _F1, :],
                          preferred_element_type=jnp.float32)
        src_col = jnp.dot(Wh, as2_ref[k], preferred_element_type=jnp.float32)
        dst_row = _dotT(ad2_ref[k], Wh)
        x2_ref[k] = _gat_head(bias_ref[...], Wh, src_col, dst_row)

    @pl.when(jnp.logical_and(phase == 1, k == _H - 1))
    def _():
        u = jnp.dot(x2_ref[0], Ws1_ref[0:_F2, :],
                    preferred_element_type=jnp.float32)
        for j in range(1, _H):
            u += jnp.dot(x2_ref[j], Ws1_ref[j * _F2:(j + 1) * _F2, :],
                         preferred_element_type=jnp.float32)
        u = jnp.tanh(u)
        scores = _dotT(u, Ws2_ref[...])
        m = jnp.max(scores, axis=0, keepdims=True)
        p = jnp.exp(scores - m)
        A = p / jnp.sum(p, axis=0, keepdims=True)
        val = jnp.float32(0.0)
        for j in range(_H):
            Mj = _dotN(A, x2_ref[j])
            val += jnp.sum(Mj * WdR_ref[:, j * _F2:(j + 1) * _F2])
        out_ref[...] = jnp.zeros((1, 8, 128), jnp.float32) + val


def kernel(h, adj, W1, a_src1, a_dst1, W2, a_src2, a_dst2, Ws1, Ws2, Wd, bd):
    B, N, F0 = h.shape
    as1 = a_src1[:, :, None]
    ad1 = a_dst1[:, None, :]
    as2 = a_src2[:, :, None]
    ad2 = a_dst2[:, None, :]
    WdR = Wd.reshape(_H, _D2)

    out = pl.pallas_call(
        _fused,
        grid=(B, 2, _H),
        in_specs=[
            pl.BlockSpec((1, N, F0), lambda b, p, k: (b, 0, 0)),
            pl.BlockSpec((1, N, N), lambda b, p, k: (b, 0, 0)),
            pl.BlockSpec((_H, F0, _F1), lambda b, p, k: (0, 0, 0)),
            pl.BlockSpec((_H, _F1, 1), lambda b, p, k: (0, 0, 0)),
            pl.BlockSpec((_H, 1, _F1), lambda b, p, k: (0, 0, 0)),
            pl.BlockSpec((_H, _D1, _F2), lambda b, p, k: (0, 0, 0)),
            pl.BlockSpec((_H, _F2, 1), lambda b, p, k: (0, 0, 0)),
            pl.BlockSpec((_H, 1, _F2), lambda b, p, k: (0, 0, 0)),
            pl.BlockSpec((_D2, _D2), lambda b, p, k: (0, 0)),
            pl.BlockSpec((_H, _D2), lambda b, p, k: (0, 0)),
            pl.BlockSpec((_H, _D2), lambda b, p, k: (0, 0)),
        ],
        out_specs=pl.BlockSpec((1, 8, 128), lambda b, p, k: (b, 0, 0)),
        out_shape=jax.ShapeDtypeStruct((B, 8, 128), jnp.float32),
        scratch_shapes=[
            pltpu.VMEM((N, N), jnp.float32),
            pltpu.VMEM((_H, N, _F1), jnp.float32),
            pltpu.VMEM((_H, N, _F2), jnp.float32),
        ],
    )(h, adj, W1, as1, ad1, W2, as2, ad2, Ws1, Ws2, WdR)
    return out[:, 0, 0] + bd[0]


# speedup vs baseline: 1.8357x; 1.0382x over previous
import jax
import jax.numpy as jnp
from jax.experimental import pallas as pl
from jax.experimental.pallas import tpu as pltpu

_B, _N, _F0 = 4, 1024, 11
_H, _F1, _F2 = 6, 32, 64
_D1 = _H * _F1
_D2 = _H * _F2


def _dotT(a, b):
    return jax.lax.dot_general(a, b, (((1,), (1,)), ((), ())),
                               preferred_element_type=jnp.float32)


def _dotN(a, b):
    return jax.lax.dot_general(a, b, (((0,), (0,)), ((), ())),
                               preferred_element_type=jnp.float32)


def _gat_head(bias, Wh, src_col, dst_row):
    e = src_col + dst_row
    e = jnp.minimum(jnp.maximum(e, 0.2 * e) + bias, 86.0)
    p = jnp.exp2(e).astype(jnp.bfloat16)
    ones = jnp.ones((_N, 1), jnp.bfloat16)
    num = jnp.dot(p, jnp.concatenate([Wh.astype(jnp.bfloat16), ones], axis=1),
                  preferred_element_type=jnp.float32)
    o = num[:, :-1] / num[:, -1:]
    return jnp.where(o > 0, o, jnp.exp(jnp.minimum(o, 0.0)) - 1.0)


def _fused(h_ref, adj_ref, W1_ref, as1_ref, ad1_ref, W2_ref, as2_ref,
           ad2_ref, Ws1_ref, Ws2_ref, WdR_ref, out_ref,
           bias_ref, x1_ref, x2_ref):
    phase = pl.program_id(1)
    k = pl.program_id(2)

    @pl.when(jnp.logical_and(phase == 0, k == 0))
    def _():
        bias_ref[...] = jnp.where(adj_ref[0] > 0.5, jnp.float32(0.0),
                                  jnp.float32(-1e9))

    @pl.when(phase == 0)
    def _():
        Wh = jnp.dot(h_ref[0], W1_ref[k], preferred_element_type=jnp.float32)
        src_col = jnp.dot(Wh, as1_ref[k], preferred_element_type=jnp.float32)
        dst_row = _dotT(ad1_ref[k], Wh)
        x1_ref[k] = _gat_head(bias_ref[...], Wh, src_col, dst_row)

    @pl.when(phase == 1)
    def _():
        Wh = jnp.dot(x1_ref[0], W2_ref[k, 0:_F1, :],
                     preferred_element_type=jnp.float32)
        for j in range(1, _H):
            Wh += jnp.dot(x1_ref[j], W2_ref[k, j * _F1:(j + 1) * _F1, :],
                          preferred_element_type=jnp.float32)
        src_col = jnp.dot(Wh, as2_ref[k], preferred_element_type=jnp.float32)
        dst_row = _dotT(ad2_ref[k], Wh)
        x2_ref[k] = _gat_head(bias_ref[...], Wh, src_col, dst_row)

    @pl.when(jnp.logical_and(phase == 1, k == _H - 1))
    def _():
        u = jnp.dot(x2_ref[0], Ws1_ref[0:_F2, :],
                    preferred_element_type=jnp.float32)
        for j in range(1, _H):
            u += jnp.dot(x2_ref[j], Ws1_ref[j * _F2:(j + 1) * _F2, :],
                         preferred_element_type=jnp.float32)
        u = jnp.tanh(u)
        scores = _dotT(u, Ws2_ref[...])
        m = jnp.max(scores, axis=0, keepdims=True)
        p = jnp.exp(scores - m)
        A = p / jnp.sum(p, axis=0, keepdims=True)
        val = jnp.float32(0.0)
        for j in range(_H):
            Mj = _dotN(A, x2_ref[j])
            val += jnp.sum(Mj * WdR_ref[:, j * _F2:(j + 1) * _F2])
        out_ref[...] = jnp.zeros((1, 8, 128), jnp.float32) + val


def kernel(h, adj, W1, a_src1, a_dst1, W2, a_src2, a_dst2, Ws1, Ws2, Wd, bd):
    B, N, F0 = h.shape
    c = jnp.float32(1.4426950408889634)
    as1 = a_src1[:, :, None] * c
    ad1 = a_dst1[:, None, :] * c
    as2 = a_src2[:, :, None] * c
    ad2 = a_dst2[:, None, :] * c
    WdR = Wd.reshape(_H, _D2)

    out = pl.pallas_call(
        _fused,
        grid=(B, 2, _H),
        in_specs=[
            pl.BlockSpec((1, N, F0), lambda b, p, k: (b, 0, 0)),
            pl.BlockSpec((1, N, N), lambda b, p, k: (b, 0, 0)),
            pl.BlockSpec((_H, F0, _F1), lambda b, p, k: (0, 0, 0)),
            pl.BlockSpec((_H, _F1, 1), lambda b, p, k: (0, 0, 0)),
            pl.BlockSpec((_H, 1, _F1), lambda b, p, k: (0, 0, 0)),
            pl.BlockSpec((_H, _D1, _F2), lambda b, p, k: (0, 0, 0)),
            pl.BlockSpec((_H, _F2, 1), lambda b, p, k: (0, 0, 0)),
            pl.BlockSpec((_H, 1, _F2), lambda b, p, k: (0, 0, 0)),
            pl.BlockSpec((_D2, _D2), lambda b, p, k: (0, 0)),
            pl.BlockSpec((_H, _D2), lambda b, p, k: (0, 0)),
            pl.BlockSpec((_H, _D2), lambda b, p, k: (0, 0)),
        ],
        out_specs=pl.BlockSpec((1, 8, 128), lambda b, p, k: (b, 0, 0)),
        out_shape=jax.ShapeDtypeStruct((B, 8, 128), jnp.float32),
        compiler_params=pltpu.CompilerParams(
            dimension_semantics=("parallel", "arbitrary", "arbitrary")),
        scratch_shapes=[
            pltpu.VMEM((N, N), jnp.float32),
            pltpu.VMEM((_H, N, _F1), jnp.float32),
            pltpu.VMEM((_H, N, _F2), jnp.float32),
        ],
    )(h, adj, W1, as1, ad1, W2, as2, ad2, Ws1, Ws2, WdR)
    return out[:, 0, 0] + bd[0]
